# baseline (speedup 1.0000x reference)
import jax
import jax.numpy as jnp
from jax import lax
from jax.experimental import pallas as pl
from jax.experimental.pallas import tpu as pltpu
from jax.experimental.pallas import tpu_sc as plsc

N_USERS = 60000
N_ITEMS = 40000
N = N_USERS + N_ITEMS
E = 1600000
EMB = 32
HALF = 16
N_LAYERS = 3

NTILES = 16
BLK = 128
BPT = 784
E_PAD = NTILES * BPT * BLK
CB = 4
CHUNKS = BPT // CB
CHUNK_E = CB * BLK

RPT = N // NTILES
ZROWS = 250
FCH = 250


def _gcn_body(cols_hbm, rows_hbm, vals_hbm, emb_hbm,
              final_hbm, t0_hbm, t1_hbm, t2_hbm, t3_hbm,
              acc, cols_a, rows_a, vals_a, gbuf_a, sem_a,
              cols_b, rows_b, vals_b, gbuf_b, sem_b):
    h = lax.axis_index("c")
    t = lax.axis_index("s")
    tables = [t0_hbm, t1_hbm, t2_hbm, t3_hbm]
    r0 = t * RPT
    cols_v = [cols_a, cols_b]
    rows_v = [rows_a, rows_b]
    vals_v = [vals_a, vals_b]
    gbuf = [gbuf_a, gbuf_b]
    sems = [sem_a, sem_b]

    for k in range(RPT // FCH):
        sub = r0 + k * FCH
        pltpu.sync_copy(emb_hbm.at[pl.ds(sub, FCH), pl.ds(h * HALF, HALF)],
                        gbuf_a.at[pl.ds(0, FCH)])
        pltpu.sync_copy(gbuf_a.at[pl.ds(0, FCH)],
                        t0_hbm.at[h].at[pl.ds(sub, FCH)])

    plsc.subcore_barrier()

    def layer(src_tbl, dst_tbl):
        zrow = jnp.zeros((HALF,), jnp.float32)

        @pl.loop(0, ZROWS)
        def _(r):
            gbuf_a[r] = zrow

        for k in range(RPT // ZROWS):
            pltpu.sync_copy(gbuf_a.at[pl.ds(0, ZROWS)],
                            acc.at[pl.ds(r0 + k * ZROWS, ZROWS)])
        plsc.subcore_barrier()

        src_half = src_tbl.at[h]

        def fetch(c, b):
            b0 = t * BPT + c * CB
            pltpu.sync_copy(cols_hbm.at[pl.ds(b0, CB)], cols_v[b])
            pltpu.sync_copy(rows_hbm.at[pl.ds(b0, CB)], rows_v[b])
            pltpu.sync_copy(vals_hbm.at[pl.ds(b0, CB)], vals_v[b])
            for j in range(CB):
                pltpu.async_copy(src_half.at[cols_v[b].at[j]],
                                 gbuf[b].at[pl.ds(j * BLK, BLK)], sems[b])

        def process(b):
            for j in range(CB):
                pltpu.make_async_copy(src_half.at[cols_v[b].at[j]],
                                      gbuf[b].at[pl.ds(j * BLK, BLK)],
                                      sems[b]).wait()
            for j in range(CB):
                @pl.loop(0, BLK // 16)
                def _(g, j=j):
                    base = j * BLK + g * 16
                    vv = vals_v[b][j, pl.ds(g * 16, 16)]
                    for i in range(16):
                        gbuf[b][base + i] = gbuf[b][base + i] * vv[i]

                pltpu.sync_copy(gbuf[b].at[pl.ds(j * BLK, BLK)],
                                acc.at[rows_v[b].at[j]], add=True)

        fetch(0, 0)

        @pl.loop(0, CHUNKS, step=2)
        def _(c):
            for b in range(2):
                cc = c + b
                nxt = cc + 1

                @pl.when(nxt < CHUNKS)
                def _():
                    fetch(nxt, (b + 1) % 2)

                process(b)

        plsc.subcore_barrier()
        pltpu.sync_copy(acc.at[pl.ds(r0, RPT)],
                        dst_tbl.at[h].at[pl.ds(r0, RPT)])
        plsc.subcore_barrier()

    layer(t0_hbm, t1_hbm)
    layer(t1_hbm, t2_hbm)
    layer(t2_hbm, t3_hbm)

    accum = gbuf_a.at[pl.ds(0, FCH)]
    lbuf = gbuf_a.at[pl.ds(FCH, FCH)]
    for k in range(RPT // FCH):
        sub = r0 + k * FCH
        pltpu.sync_copy(tables[0].at[h].at[pl.ds(sub, FCH)], accum)
        for l in range(1, N_LAYERS + 1):
            pltpu.sync_copy(tables[l].at[h].at[pl.ds(sub, FCH)], lbuf)

            @pl.loop(0, FCH)
            def _(r):
                accum[r] = accum[r] + lbuf[r]

        @pl.loop(0, FCH)
        def _(r):
            accum[r] = accum[r] * 0.25

        pltpu.sync_copy(accum,
                        final_hbm.at[pl.ds(sub, FCH), pl.ds(h * HALF, HALF)])


@jax.jit
def _gcn(cols2d, rows2d, vals2d, all_emb):
    mesh = plsc.VectorSubcoreMesh(core_axis_name="c", subcore_axis_name="s")
    f32 = jnp.float32
    out_type = (
        jax.ShapeDtypeStruct((N, EMB), f32),
        jax.ShapeDtypeStruct((2, N, HALF), f32),
        jax.ShapeDtypeStruct((2, N, HALF), f32),
        jax.ShapeDtypeStruct((2, N, HALF), f32),
        jax.ShapeDtypeStruct((2, N, HALF), f32),
    )
    bufset = lambda: [
        pltpu.VMEM((CB, BLK), jnp.int32),
        pltpu.VMEM((CB, BLK), jnp.int32),
        pltpu.VMEM((CB, BLK), f32),
        pltpu.VMEM((CHUNK_E, HALF), f32),
        pltpu.SemaphoreType.DMA,
    ]
    scratch = [pltpu.VMEM_SHARED((N, HALF), f32)] + bufset() + bufset()
    run = pl.kernel(_gcn_body, out_type=out_type, mesh=mesh,
                    scratch_types=scratch,
                    compiler_params=pltpu.CompilerParams(
                        use_tc_tiling_on_sc=False))
    return run(cols2d, rows2d, vals2d, all_emb)


def kernel(adj_rows, adj_cols, adj_vals, user_emb, item_emb):
    all_emb = jnp.concatenate([user_emb, item_emb], axis=0)
    pad = E_PAD - E
    cols2d = jnp.pad(adj_cols, (0, pad)).reshape(E_PAD // BLK, BLK)
    rows2d = jnp.pad(adj_rows, (0, pad)).reshape(E_PAD // BLK, BLK)
    vals2d = jnp.pad(adj_vals, (0, pad)).reshape(E_PAD // BLK, BLK)
    final, _, _, _, _ = _gcn(cols2d, rows2d, vals2d, all_emb)
    return (final[:N_USERS], final[N_USERS:])
